# baseline (speedup 1.0000x reference)
import functools
import math

import jax
import jax.numpy as jnp
from jax import lax
from jax.experimental import pallas as pl
from jax.experimental.pallas import tpu as pltpu
from jax.experimental.pallas import tpu_sc as plsc

EMB = 64
SCALE = 8.0

NC = 2
NS = 16
NW = NC * NS
LANES = 16

B_TOTAL = 4096 * 200
ROWS_PER_W = B_TOTAL // NW
CHUNK = 128
CHUNKS_PER_GROUP = 4
GROUP_ROWS = CHUNK * CHUNKS_PER_GROUP
N_GROUPS = ROWS_PER_W // GROUP_ROWS
N_CHUNKS = ROWS_PER_W // CHUNK

_mesh = plsc.VectorSubcoreMesh(
    core_axis_name="c", subcore_axis_name="s", num_cores=NC, num_subcores=NS)


@functools.partial(
    pl.kernel,
    out_type=jax.ShapeDtypeStruct((B_TOTAL, EMB), jnp.float32),
    mesh=_mesh,
    scratch_types=[
        pltpu.VMEM((N_CHUNKS, CHUNK), jnp.int32),
        pltpu.VMEM((GROUP_ROWS, EMB), jnp.float32),
        pltpu.SemaphoreType.DMA,
        pltpu.SemaphoreType.DMA,
    ],
    compiler_params=pltpu.CompilerParams(use_tc_tiling_on_sc=False),
)
def _emb_lookup(tok_hbm, table_hbm, out_hbm, idx_v, buf, gsem, osem):
    wid = lax.axis_index("s") * NC + lax.axis_index("c")
    pltpu.sync_copy(tok_hbm.at[pl.ds(wid * N_CHUNKS, N_CHUNKS)], idx_v)
    row_base = wid * ROWS_PER_W

    def group_body(g, carry):
        copies = []
        for j in range(CHUNKS_PER_GROUP):
            copies.append(pltpu.async_copy(
                table_hbm.at[idx_v.at[g * CHUNKS_PER_GROUP + j]],
                buf.at[pl.ds(j * CHUNK, CHUNK)], gsem))
        for cp in copies:
            cp.wait()

        def scale_body(r, c):
            for l in range(EMB // LANES):
                sl = pl.ds(l * LANES, LANES)
                buf[r, sl] = buf[r, sl] * SCALE
            return c

        lax.fori_loop(0, GROUP_ROWS, scale_body, 0, unroll=4)

        pltpu.async_copy(
            buf, out_hbm.at[pl.ds(row_base + g * GROUP_ROWS, GROUP_ROWS)],
            osem).wait()
        return carry

    lax.fori_loop(0, N_GROUPS, group_body, 0)


def kernel(tokens, embedding):
    tok = tokens.reshape(B_TOTAL // CHUNK, CHUNK).astype(jnp.int32)
    out = _emb_lookup(tok, embedding)
    return out.reshape(tokens.shape[0], tokens.shape[1], EMB)


# speedup vs baseline: 1.0709x; 1.0709x over previous
import functools
import math

import jax
import jax.numpy as jnp
from jax import lax
from jax.experimental import pallas as pl
from jax.experimental.pallas import tpu as pltpu
from jax.experimental.pallas import tpu_sc as plsc

EMB = 64
SCALE = 8.0
LANES = 16

NC = 2
NS = 16
NW = NC * NS

B_TOTAL = 4096 * 200
ROWS_PER_W = B_TOTAL // NW
CHUNK = 128
CHUNKS_PER_GROUP = 2
GROUP_ROWS = CHUNK * CHUNKS_PER_GROUP
N_GROUPS = ROWS_PER_W // GROUP_ROWS
N_CHUNKS = ROWS_PER_W // CHUNK
NBUF = 4

_mesh = plsc.VectorSubcoreMesh(
    core_axis_name="c", subcore_axis_name="s", num_cores=NC, num_subcores=NS)


@functools.partial(
    pl.kernel,
    out_type=jax.ShapeDtypeStruct((B_TOTAL, EMB), jnp.float32),
    mesh=_mesh,
    scratch_types=[
        pltpu.VMEM((N_CHUNKS, CHUNK), jnp.int32),
    ] + [pltpu.VMEM((GROUP_ROWS, EMB), jnp.float32) for _ in range(NBUF)]
      + [pltpu.SemaphoreType.DMA for _ in range(2 * NBUF)],
    compiler_params=pltpu.CompilerParams(use_tc_tiling_on_sc=False),
)
def _emb_lookup(tok_hbm, table_hbm, out_hbm, idx_v,
                b0, b1, b2, b3, g0, g1, g2, g3, o0, o1, o2, o3):
    bufs = [b0, b1, b2, b3]
    gsems = [g0, g1, g2, g3]
    osems = [o0, o1, o2, o3]

    wid = lax.axis_index("s") * NC + lax.axis_index("c")
    pltpu.sync_copy(tok_hbm.at[pl.ds(wid * N_CHUNKS, N_CHUNKS)], idx_v)
    row_base = wid * ROWS_PER_W

    def fire_gather(g, j):
        for c in range(CHUNKS_PER_GROUP):
            pltpu.async_copy(
                table_hbm.at[idx_v.at[g * CHUNKS_PER_GROUP + c]],
                bufs[j].at[pl.ds(c * CHUNK, CHUNK)], gsems[j])

    def wait_gather(j):
        for c in range(CHUNKS_PER_GROUP):
            pltpu.make_async_copy(
                table_hbm.at[idx_v.at[0]],
                bufs[j].at[pl.ds(c * CHUNK, CHUNK)], gsems[j]).wait()

    def out_slice(g):
        return out_hbm.at[pl.ds(row_base + g * GROUP_ROWS, GROUP_ROWS)]

    def fire_out(g, j):
        pltpu.async_copy(bufs[j], out_slice(g), osems[j])

    def wait_out(j):
        pltpu.make_async_copy(bufs[j], out_slice(0), osems[j]).wait()

    def scale(j):
        buf = bufs[j]

        def body(r, carry):
            for l in range(EMB // LANES):
                sl = pl.ds(l * LANES, LANES)
                buf[r, sl] = buf[r, sl] * SCALE
            return carry

        lax.fori_loop(0, GROUP_ROWS, body, 0, unroll=8)

    fire_gather(0, 0)
    fire_gather(1, 1)

    for g in (0, 1):
        fire_gather(g + 2, (g + 2) % NBUF)
        wait_gather(g % NBUF)
        scale(g % NBUF)
        fire_out(g, g % NBUF)

    def loop_body(t, carry):
        for jj in range(NBUF):
            g = NBUF * t + 2 + jj
            j = (2 + jj) % NBUF
            wait_out(jj)
            fire_gather(g + 2, jj)
            wait_gather(j)
            scale(j)
            fire_out(g, j)
        return carry

    lax.fori_loop(0, (N_GROUPS - NBUF) // NBUF, loop_body, 0)

    for g in (N_GROUPS - 2, N_GROUPS - 1):
        wait_out((g + 2) % NBUF)
        wait_gather(g % NBUF)
        scale(g % NBUF)
        fire_out(g, g % NBUF)

    wait_out((N_GROUPS - 2) % NBUF)
    wait_out((N_GROUPS - 1) % NBUF)


def kernel(tokens, embedding):
    tok = tokens.reshape(B_TOTAL // CHUNK, CHUNK).astype(jnp.int32)
    out = _emb_lookup(tok, embedding)
    return out.reshape(tokens.shape[0], tokens.shape[1], EMB)
